# baseline (speedup 1.0000x reference)
import jax
import jax.numpy as jnp
from jax.experimental import pallas as pl
from jax.experimental.pallas import tpu as pltpu

_STEPS = 100
_BR = 16


def _fma_body(t_ref, sac_ref, somac_ref, x_ref, n_ref, o_ref):
    tcol = t_ref[...]
    steps = jax.lax.broadcasted_iota(jnp.int32, (_BR, _STEPS), 1)
    m = tcol == steps
    c1 = jnp.sum(jnp.where(m, sac_ref[...], 0.0), axis=1, keepdims=True)
    c2 = jnp.sum(jnp.where(m, somac_ref[...], 0.0), axis=1, keepdims=True)
    o_ref[...] = c1 * x_ref[...] + c2 * n_ref[...]


def kernel(x_start, noise, sqrt_alphas_cumprod, sqrt_one_minus_alphas_cumprod, t):
    B, V = x_start.shape
    grid = (B // _BR,)
    t2 = t.reshape(B, 1)
    sac2 = sqrt_alphas_cumprod.reshape(1, _STEPS)
    somac2 = sqrt_one_minus_alphas_cumprod.reshape(1, _STEPS)

    return pl.pallas_call(
        _fma_body,
        grid=grid,
        in_specs=[
            pl.BlockSpec((_BR, 1), lambda i: (i, 0)),
            pl.BlockSpec((1, _STEPS), lambda i: (0, 0)),
            pl.BlockSpec((1, _STEPS), lambda i: (0, 0)),
            pl.BlockSpec((_BR, V), lambda i: (i, 0)),
            pl.BlockSpec((_BR, V), lambda i: (i, 0)),
        ],
        out_specs=pl.BlockSpec((_BR, V), lambda i: (i, 0)),
        out_shape=jax.ShapeDtypeStruct((B, V), x_start.dtype),
        compiler_params=pltpu.CompilerParams(
            dimension_semantics=("arbitrary",),
        ),
    )(t2, sac2, somac2, x_start, noise)


# speedup vs baseline: 1.0002x; 1.0002x over previous
import jax
import jax.numpy as jnp
from jax.experimental import pallas as pl
from jax.experimental.pallas import tpu as pltpu

_STEPS = 100
_BR = 16


def _fma_body(t_ref, sac_ref, somac_ref, x_ref, n_ref, o_ref):
    tcol = t_ref[...]
    steps = jax.lax.broadcasted_iota(jnp.int32, (_BR, _STEPS), 1)
    m = tcol == steps
    c1 = jnp.sum(jnp.where(m, sac_ref[...], 0.0), axis=1, keepdims=True)
    c2 = jnp.sum(jnp.where(m, somac_ref[...], 0.0), axis=1, keepdims=True)
    o_ref[...] = c1 * x_ref[...] + c2 * n_ref[...]


def kernel(x_start, noise, sqrt_alphas_cumprod, sqrt_one_minus_alphas_cumprod, t):
    B, V = x_start.shape
    grid = (B // _BR,)
    t2 = t.reshape(B, 1)
    sac2 = sqrt_alphas_cumprod.reshape(1, _STEPS)
    somac2 = sqrt_one_minus_alphas_cumprod.reshape(1, _STEPS)

    return pl.pallas_call(
        _fma_body,
        grid=grid,
        in_specs=[
            pl.BlockSpec((_BR, 1), lambda i: (i, 0)),
            pl.BlockSpec((1, _STEPS), lambda i: (0, 0)),
            pl.BlockSpec((1, _STEPS), lambda i: (0, 0)),
            pl.BlockSpec((_BR, V), lambda i: (i, 0)),
            pl.BlockSpec((_BR, V), lambda i: (i, 0)),
        ],
        out_specs=pl.BlockSpec((_BR, V), lambda i: (i, 0)),
        out_shape=jax.ShapeDtypeStruct((B, V), x_start.dtype),
        compiler_params=pltpu.CompilerParams(
            dimension_semantics=("parallel",),
        ),
    )(t2, sac2, somac2, x_start, noise)
